# Initial kernel scaffold; baseline (speedup 1.0000x reference)
import jax
import jax.numpy as jnp
from jax.experimental import pallas as pl


def kernel(x, W):
    raise NotImplementedError("write your pallas kernel here")


# speedup vs baseline: 80.6017x; 80.6017x over previous
import jax
import jax.numpy as jnp
from jax.experimental import pallas as pl
from jax.experimental.pallas import tpu as pltpu

IN_F = 1024
OUT_F = 1024
KSEL = 16
BATCH = 256
CHUNKS = IN_F // KSEL


def _oddeven_pairs(n):
    pairs = []

    def merge(lo, m, r):
        step = r * 2
        if step < m:
            merge(lo, m, step)
            merge(lo + r, m, step)
            for i in range(lo + r, lo + m - r, step):
                pairs.append((i, i + r))
        else:
            pairs.append((lo, lo + r))

    def sort(lo, m):
        if m > 1:
            half = m // 2
            sort(lo, half)
            sort(lo + half, half)
            merge(lo, m, 1)

    sort(0, n)
    return pairs


_SORT16 = _oddeven_pairs(KSEL)


def _sort_desc(planes):
    planes = list(planes)
    for i, j in _SORT16:
        a, b = planes[i], planes[j]
        planes[i] = jnp.maximum(a, b)
        planes[j] = jnp.minimum(a, b)
    return planes


def _bitonic_merge(planes, descending):
    planes = list(planes)
    for d in (8, 4, 2, 1):
        for i in range(KSEL):
            if i & d == 0:
                a, b = planes[i], planes[i + d]
                if descending:
                    planes[i] = jnp.maximum(a, b)
                    planes[i + d] = jnp.minimum(a, b)
                else:
                    planes[i] = jnp.minimum(a, b)
                    planes[i + d] = jnp.maximum(a, b)
    return planes


def _tree_sum(planes):
    vals = list(planes)
    while len(vals) > 1:
        nxt = [vals[i] + vals[i + 1] for i in range(0, len(vals) - 1, 2)]
        if len(vals) % 2:
            nxt.append(vals[-1])
        vals = nxt
    return vals[0]


def _body(x_ref, wt_ref, out_ref):
    neg = jnp.full((8, 128), -jnp.inf, jnp.float32)
    pos = jnp.full((8, 128), jnp.inf, jnp.float32)

    def chunk_step(c, carry):
        top, bot, tot = carry
        base = c * KSEL
        w16 = wt_ref[pl.ds(base, KSEL), :, :]
        planes = [x_ref[0, 0, base + p] * w16[p] for p in range(KSEL)]
        tot = tot + _tree_sum(planes)
        sp = _sort_desc(planes)
        ctop = [jnp.maximum(top[p], sp[KSEL - 1 - p]) for p in range(KSEL)]
        top = tuple(_bitonic_merge(ctop, descending=True))
        cbot = [jnp.minimum(bot[p], sp[p]) for p in range(KSEL)]
        bot = tuple(_bitonic_merge(cbot, descending=False))
        return top, bot, tot

    init = ((neg,) * KSEL, (pos,) * KSEL, jnp.zeros((8, 128), jnp.float32))
    top, bot, tot = jax.lax.fori_loop(0, CHUNKS, chunk_step, init)
    out_ref[0] = tot - _tree_sum(top) + _tree_sum(bot)


@jax.jit
def kernel(x, W):
    wt = W.T.reshape(IN_F, 8, 128)
    x3 = x.reshape(BATCH, 1, IN_F)
    out3 = pl.pallas_call(
        _body,
        grid=(BATCH,),
        in_specs=[
            pl.BlockSpec((1, 1, IN_F), lambda b: (b, 0, 0),
                         memory_space=pltpu.SMEM),
            pl.BlockSpec((IN_F, 8, 128), lambda b: (0, 0, 0)),
        ],
        out_specs=pl.BlockSpec((1, 8, 128), lambda b: (b, 0, 0)),
        out_shape=jax.ShapeDtypeStruct((BATCH, 8, 128), jnp.float32),
    )(x3, wt)
    return out3.reshape(BATCH, OUT_F)
